# Initial kernel scaffold; baseline (speedup 1.0000x reference)
import jax
import jax.numpy as jnp
from jax.experimental import pallas as pl


def kernel(X_persis, diagram_slices, mu0, log_mu1, log_sigma0, log_sigma1):
    raise NotImplementedError("write your pallas kernel here")


# speedup vs baseline: 4.0216x; 4.0216x over previous
import functools
import math

import jax
import jax.numpy as jnp
from jax.experimental import pallas as pl

_N_POINTS = 32768
_N_UNITS = 1024
_N_DIAG = 16
_THRESH = 0.01
_BLOCK_R = 2048
_GRID = _N_POINTS // _BLOCK_R

_C45 = math.cos(-math.pi / 4.0)
_S45 = math.sin(-math.pi / 4.0)


def _topo_kernel(x_ref, sl_ref, mu0_ref, lmu1_ref, ls0_ref, ls1_ref, out_ref):
    step = pl.program_id(0)

    mu0 = mu0_ref[:]
    mu1 = jnp.exp(lmu1_ref[:])
    s0 = jnp.exp(ls0_ref[:])
    s1 = jnp.exp(ls1_ref[:])

    x = x_ref[:]
    xa = x[:, 0]
    xb = x[:, 1]
    c = jnp.float32(_C45)
    s = jnp.float32(_S45)
    x0 = xa * c - xb * s
    x1 = xa * s + xb * c

    thresh = jnp.float32(_THRESH)
    x1_alt = jnp.log(x1 / thresh) * thresh + thresh
    x1e = jnp.where(x0 >= thresh, x1, x1_alt)

    a = x0[:, None] * s0[None, :] - (s0 * mu0)[None, :]
    b = x1e[:, None] * s1[None, :] - (s1 * mu1)[None, :]
    out = jnp.exp(-(a * a + b * b))

    gi = step * _BLOCK_R + jax.lax.broadcasted_iota(jnp.int32, (_N_DIAG, _BLOCK_R), 1)
    st = sl_ref[:, 0:1]
    en = sl_ref[:, 1:2]
    w = ((gi >= st) & (gi < en)).astype(jnp.float32)

    contrib = jax.lax.dot(w, out, precision=jax.lax.Precision.HIGHEST,
                          preferred_element_type=jnp.float32)

    @pl.when(step == 0)
    def _():
        out_ref[:] = contrib

    @pl.when(step != 0)
    def _():
        out_ref[:] = out_ref[:] + contrib


@jax.jit
def kernel(X_persis, diagram_slices, mu0, log_mu1, log_sigma0, log_sigma1):
    sl = diagram_slices.astype(jnp.int32)
    return pl.pallas_call(
        _topo_kernel,
        grid=(_GRID,),
        in_specs=[
            pl.BlockSpec((_BLOCK_R, 2), lambda i: (i, 0)),
            pl.BlockSpec((_N_DIAG, 2), lambda i: (0, 0)),
            pl.BlockSpec((_N_UNITS,), lambda i: (0,)),
            pl.BlockSpec((_N_UNITS,), lambda i: (0,)),
            pl.BlockSpec((_N_UNITS,), lambda i: (0,)),
            pl.BlockSpec((_N_UNITS,), lambda i: (0,)),
        ],
        out_specs=pl.BlockSpec((_N_DIAG, _N_UNITS), lambda i: (0, 0)),
        out_shape=jax.ShapeDtypeStruct((_N_DIAG, _N_UNITS), jnp.float32),
    )(X_persis, sl, mu0, log_mu1, log_sigma0, log_sigma1)
